# baseline (speedup 1.0000x reference)
import functools

import jax
import jax.numpy as jnp
from jax.experimental import pallas as pl
from jax.experimental.pallas import tpu as pltpu

BATCH = 8
IMG = 512
FPN_CH = 64
NUM_CLASSES = 8
LVL_HW = ((64, 64), (32, 32), (16, 16), (8, 8), (4, 4))


def _cp(vmem, n=1):
    return pltpu.CompilerParams(
        dimension_semantics=("parallel",) * n,
        vmem_limit_bytes=vmem)


_RS = ((1, 0), (0, 1), (1, 1))


def _conv3x3_s2(x, w_ref, b_ref, C):
    H = x.shape[0]
    oh = H // 2
    ph = x.reshape(oh, 2, oh, 2, C)
    PP = [[jnp.pad(ph[:, a, :, bb, :], ((1, 0), (1, 0), (0, 0)))
           for bb in range(2)] for a in range(2)]
    taps = []
    for p in range(3):
        a, sr = _RS[p]
        for q in range(3):
            bb, sc = _RS[q]
            taps.append(PP[a][bb][sr:sr + oh, sc:sc + oh, :])
    a2 = jnp.concatenate(taps, axis=-1).reshape(oh * oh, 9 * C)
    acc = jnp.dot(a2, w_ref[...], preferred_element_type=jnp.float32)
    acc = acc + b_ref[...]
    acc = acc * jax.nn.sigmoid(acc)
    return acc.astype(jnp.bfloat16).reshape(oh, oh, -1)


def _sep_node(xs, fuse, dw, pw, pb, pre_swish, post_act):
    H, W, C = xs[0].shape
    if fuse is not None:
        wk = [jnp.maximum(f, 0.0) for f in fuse]
        inv = 1.0 / (sum(wk) + 1e-4)
        s = (wk[0] * inv) * xs[0]
        for k in range(1, len(xs)):
            s = s + (wk[k] * inv) * xs[k]
    else:
        s = xs[0]
    if pre_swish:
        s = s * jax.nn.sigmoid(s)
    sp = jnp.pad(s, ((1, 1), (1, 1), (0, 0)))
    acc = None
    for k in range(9):
        dy, dx = divmod(k, 3)
        t = sp[dy:dy + H, dx:dx + W, :] * dw[k]
        acc = t if acc is None else acc + t
    out = jnp.dot(acc.reshape(H * W, C).astype(jnp.bfloat16), pw,
                  preferred_element_type=jnp.float32) + pb
    if post_act == "swish":
        out = out * jax.nn.sigmoid(out)
    elif post_act == "sigmoid":
        out = jax.nn.sigmoid(out)
    return out.reshape(H, W, -1)


def _pool(x):
    H, W, C = x.shape
    oh, ow = H // 2, W // 2
    ph = x.reshape(oh, 2, ow, 2, C)
    PP = [[jnp.pad(ph[:, a, :, bb, :],
                   ((0, 1), (0, 1), (0, 0)), constant_values=-jnp.inf)
           for bb in range(2)] for a in range(2)]
    RS = ((0, 0), (1, 0), (0, 1))
    o = None
    for p in range(3):
        a, sr = RS[p]
        for q in range(3):
            bb, sc = RS[q]
            t = PP[a][bb][sr:sr + oh, sc:sc + ow, :]
            o = t if o is None else jnp.maximum(o, t)
    return o


def _up(x):
    return jnp.repeat(jnp.repeat(x, 2, axis=0), 2, axis=1)


def _c1_kernel(xm_ref, xh_ref, w_ref, b_ref, o_ref):
    t = pl.program_id(1)
    halo = xh_ref[0]
    halo = jnp.where(t > 0, halo, jnp.zeros_like(halo))
    rows = jnp.concatenate([halo, xm_ref[0]], axis=0)
    rows = jnp.pad(rows, ((0, 0), (1, 0), (0, 0)))
    taps = [rows[di:di + 32, dj:dj + 128, :]
            for di in range(2) for dj in range(2)]
    a2 = jnp.concatenate(taps, axis=-1).reshape(32 * 128, 192)
    acc = jnp.dot(a2, w_ref[...], preferred_element_type=jnp.float32)
    acc = acc + b_ref[...]
    acc = acc * jax.nn.sigmoid(acc)
    o_ref[0] = acc.astype(jnp.bfloat16).reshape(32, 128, 128)


def _bb2_kernel(c_ref, w2_ref, b2_ref, w3_ref, b3_ref,
                w4_ref, b4_ref, w5_ref, b5_ref, p3_ref, p4_ref, p5_ref):
    cc = c_ref[0]
    pp = jnp.pad(cc, ((1, 0), (1, 0), (0, 0)))
    taps = []
    for p in range(3):
        a, sr = _RS[p]
        for q in range(3):
            bb, sc = _RS[q]
            g = (a * 2 + bb) * 32
            taps.append(pp[sr:sr + 128, sc:sc + 128, g:g + 32])
    a2 = jnp.concatenate(taps, axis=-1).reshape(128 * 128, 288)
    acc = jnp.dot(a2, w2_ref[...], preferred_element_type=jnp.float32)
    acc = acc + b2_ref[...]
    acc = acc * jax.nn.sigmoid(acc)
    c2 = acc.astype(jnp.bfloat16).reshape(128, 128, 24)
    p3 = _conv3x3_s2(c2, w3_ref, b3_ref, 24)
    p4 = _conv3x3_s2(p3, w4_ref, b4_ref, 40)
    p5 = _conv3x3_s2(p4, w5_ref, b5_ref, 112)
    p3_ref[0] = p3
    p4_ref[0] = p4
    p5_ref[0] = p5


def _bifpn_kernel(p3_ref, p4_ref, p5_ref, fw_ref,
                  p3dw_ref, p3db_ref, p4dw_ref, p4db_ref,
                  p5dw_ref, p5db_ref, p56w_ref, p56b_ref,
                  dw_ref, pw_ref, pb_ref,
                  o3_ref, o4_ref, o5_ref, o6_ref, o7_ref):
    def dense(x, w_ref2, b_ref2):
        H, W, C = x.shape
        o = jnp.dot(x.reshape(H * W, C), w_ref2[...],
                    preferred_element_type=jnp.float32) + b_ref2[...]
        return o.reshape(H, W, -1)

    p3_in = dense(p3_ref[0], p3dw_ref, p3db_ref)
    p4b = dense(p4_ref[0], p4dw_ref, p4db_ref)
    p5b = dense(p5_ref[0], p5dw_ref, p5db_ref)
    p5p6 = dense(p5_ref[0], p56w_ref, p56b_ref)
    p6_in = _pool(p5p6)
    p7_in = _pool(p6_in)

    f3 = f4 = f5 = f6 = f7 = None
    for c in range(3):
        base = 8 * c

        def F(off, n, c=c):
            return [fw_ref[19 * c + off + j] for j in range(n)]

        def node(xs, fuse, k, base=base):
            return _sep_node(xs, fuse, dw_ref[base + k], pw_ref[base + k],
                             pb_ref[base + k], True, "none")

        if c == 0:
            p4_1, p4_2 = p4b[..., :FPN_CH], p4b[..., FPN_CH:]
            p5_1, p5_2 = p5b[..., :FPN_CH], p5b[..., FPN_CH:]
            p3a, p6a, p7a = p3_in, p6_in, p7_in
        else:
            p3a, p4_1, p5_1, p6a, p7a = f3, f4, f5, f6, f7
            p4_2, p5_2 = p4_1, p5_1
        p6_td = node([p6a, _up(p7a)], F(0, 2), 0)
        p5_td = node([p5_1, _up(p6_td)], F(2, 2), 1)
        p4_td = node([p4_1, _up(p5_td)], F(4, 2), 2)
        f3 = node([p3a, _up(p4_td)], F(6, 2), 3)
        f4 = node([p4_2, p4_td, _pool(f3)], F(8, 3), 4)
        f5 = node([p5_2, p5_td, _pool(f4)], F(11, 3), 5)
        f6 = node([p6a, p6_td, _pool(f5)], F(14, 3), 6)
        f7 = node([p7a, _pool(f6)], F(17, 2), 7)

    o3_ref[0] = f3
    o4_ref[0] = f4
    o5_ref[0] = f5
    o6_ref[0] = f6
    o7_ref[0] = f7


_LVL_OFF = (0, 4096, 5120, 5376, 5440)


def _roll(x, k):
    return jnp.concatenate([x[:, -k:], x[:, :-k]], axis=1)


def _heads_kernel(f3_ref, f4_ref, f5_ref, f6_ref, f7_ref, base_ref, scal_ref,
                  rldw_ref, rlpw_ref, rlpb_ref, rhdw_ref, rhpw_ref, rhpb_ref,
                  cldw_ref, clpw_ref, clpb_ref, chdw_ref, chpw_ref, chpb_ref,
                  bx_ref, cl_ref):
    frefs = (f3_ref, f4_ref, f5_ref, f6_ref, f7_ref)
    for lvl in range(5):
        f = frefs[lvl][0]
        H, W, _ = f.shape
        r = f
        for l in range(3):
            r = _sep_node([r], None, rldw_ref[l], rlpw_ref[l], rlpb_ref[l],
                          False, "swish")
        r = _sep_node([r], None, rhdw_ref[...], rhpw_ref[...], rhpb_ref[...],
                      False, "none")
        R = r.reshape(H * W, 36)
        off = _LVL_OFF[lvl]
        sb = base_ref[off:off + H * W, :]
        sc = scal_ref[off:off + H * W, :]
        rm1, rp1 = _roll(R, -1), _roll(R, 1)
        rm3, rp3 = _roll(R, -3), _roll(R, 3)
        lane = jax.lax.broadcasted_iota(jnp.int32, (H * W, 36), 1) % 4
        dpos = jnp.where(lane == 0, rm1,
                         jnp.where(lane == 3, rp3, rp1))
        dsz = jnp.where(lane == 0, rm3,
                        jnp.where(lane == 3, rp1, rm1))
        center = dpos * sc + sb
        half = jnp.exp(dsz) * sc * 0.5
        lo = jnp.maximum(center - half, 0.0)
        hi = jnp.minimum(center + half, IMG - 1.0)
        bx_ref[0, off:off + H * W, :] = jnp.where(lane < 2, lo, hi)
        c = f
        for l in range(3):
            c = _sep_node([c], None, cldw_ref[l], clpw_ref[l], clpb_ref[l],
                          False, "swish")
        c = _sep_node([c], None, chdw_ref[...], chpw_ref[...], chpb_ref[...],
                      False, "sigmoid")
        cl_ref[0, off:off + H * W, :] = c.reshape(H * W, 72)


def _conv1_w192(w1):
    rows, grps, srcs = [], [], []
    for al in range(2):
        for p in range(3):
            di, a4 = (0, 3) if (al == 0 and p == 0) else (1, 2 * al + p - 1)
            for be in range(2):
                for q in range(3):
                    dj, b4 = ((0, 3) if (be == 0 and q == 0)
                              else (1, 2 * be + q - 1))
                    for cch in range(3):
                        rows.append((di * 2 + dj) * 48 + a4 * 12 + b4 * 3
                                    + cch)
                        grps.append(al * 2 + be)
                        srcs.append((p * 3 + q) * 3 + cch)
    w4 = jnp.zeros((192, 4, 32), jnp.float32)
    w4 = w4.at[jnp.asarray(rows), jnp.asarray(grps)].set(w1[jnp.asarray(srcs)])
    return w4.reshape(192, 128)


def _full_spec(shape):
    nd = len(shape)
    return pl.BlockSpec(shape, lambda b, nd=nd: (0,) * nd)


def _batch_spec(shape):
    nd = len(shape)
    return pl.BlockSpec((1,) + tuple(shape), lambda b, nd=nd: (b,) + (0,) * nd)


def _half_pipeline(ws, x, base36, scal36, w192, b128):
    bf = jnp.bfloat16
    f32 = jnp.float32
    B = x.shape[0]

    xs = jnp.transpose(x.astype(bf).reshape(B, 3, 128, 4, 128, 4),
                       (0, 2, 4, 3, 5, 1)).reshape(B, 128, 128, 48)
    c1 = pl.pallas_call(
        _c1_kernel,
        out_shape=jax.ShapeDtypeStruct((B, 128, 128, 128), bf),
        grid=(B, 4),
        in_specs=[
            pl.BlockSpec((1, 32, 128, 48), lambda b, t: (b, t, 0, 0)),
            pl.BlockSpec((1, 1, 128, 48),
                         lambda b, t: (b, jnp.maximum(32 * t - 1, 0), 0, 0)),
            pl.BlockSpec((192, 128), lambda b, t: (0, 0)),
            pl.BlockSpec((1, 128), lambda b, t: (0, 0))],
        out_specs=pl.BlockSpec((1, 32, 128, 128), lambda b, t: (b, t, 0, 0)),
        compiler_params=_cp(40 * 1024 * 1024, 2),
    )(xs, xs, w192, b128)

    bb_args = [c1]
    bb_specs = [_batch_spec((128, 128, 128))]
    for wi, bi in ((3, 2), (5, 4), (7, 6), (9, 8)):
        bb_args += [ws[wi].astype(bf), ws[bi].reshape(1, -1)]
        bb_specs += [_full_spec(ws[wi].shape), _full_spec((1, ws[bi].shape[0]))]
    p3, p4, p5 = pl.pallas_call(
        _bb2_kernel,
        out_shape=(jax.ShapeDtypeStruct((B, 64, 64, 40), bf),
                   jax.ShapeDtypeStruct((B, 32, 32, 112), bf),
                   jax.ShapeDtypeStruct((B, 16, 16, 320), bf)),
        grid=(B,),
        in_specs=bb_specs,
        out_specs=(_batch_spec((64, 64, 40)), _batch_spec((32, 32, 112)),
                   _batch_spec((16, 16, 320))),
        compiler_params=_cp(60 * 1024 * 1024),
    )(*bb_args)

    sep_off = {"conv3_up": 0, "conv4_down": 3, "conv4_up": 6, "conv5_down": 9,
               "conv5_up": 12, "conv6_down": 15, "conv6_up": 18,
               "conv7_down": 21}
    node_order = ("conv6_up", "conv5_up", "conv4_up", "conv3_up",
                  "conv4_down", "conv5_down", "conv6_down", "conv7_down")
    fuse_off = {"w3_up": 24, "w4_dn": 25, "w4_up": 26, "w5_dn": 27,
                "w5_up": 28, "w6_dn": 29, "w6_up": 30, "w7_dn": 31}
    fuse_order = ("w6_up", "w5_up", "w4_up", "w3_up", "w4_dn", "w5_dn",
                  "w6_dn", "w7_dn")
    DW, PW, PB, FW = [], [], [], []
    for c in range(3):
        base = 10 + 32 * c
        for name in node_order:
            o = base + sep_off[name]
            DW.append(ws[o])
            PB.append(ws[o + 1])
            PW.append(ws[o + 2])
        for name in fuse_order:
            FW.append(ws[base + fuse_off[name]])
    DW = jnp.stack(DW)
    PW = jnp.stack(PW).astype(bf)
    PB = jnp.stack(PB)
    FW = jnp.concatenate(FW)

    fp_args = [p3, p4, p5, FW,
               ws[107].astype(bf), ws[106].reshape(1, 64),
               ws[109].astype(bf), ws[108].reshape(1, 128),
               ws[111].astype(bf), ws[110].reshape(1, 128),
               ws[113].astype(bf), ws[112].reshape(1, 64),
               DW, PW, PB]
    fp_specs = [_batch_spec((64, 64, 40)), _batch_spec((32, 32, 112)),
                _batch_spec((16, 16, 320)),
                pl.BlockSpec(memory_space=pltpu.MemorySpace.SMEM),
                _full_spec((40, 64)), _full_spec((1, 64)),
                _full_spec((112, 128)), _full_spec((1, 128)),
                _full_spec((320, 128)), _full_spec((1, 128)),
                _full_spec((320, 64)), _full_spec((1, 64)),
                _full_spec((24, 9, 64)), _full_spec((24, 64, 64)),
                _full_spec((24, 64))]
    feats = pl.pallas_call(
        _bifpn_kernel,
        out_shape=tuple(jax.ShapeDtypeStruct((B, h, w, FPN_CH), f32)
                        for h, w in LVL_HW),
        grid=(B,),
        in_specs=fp_specs,
        out_specs=tuple(_batch_spec((h, w, FPN_CH)) for h, w in LVL_HW),
        compiler_params=_cp(40 * 1024 * 1024),
    )(*fp_args)

    rldw = jnp.stack([ws[129], ws[132], ws[135]])
    rlpb = jnp.stack([ws[130], ws[133], ws[136]])
    rlpw = jnp.stack([ws[131], ws[134], ws[137]]).astype(bf)
    cldw = jnp.stack([ws[117], ws[120], ws[123]])
    clpb = jnp.stack([ws[118], ws[121], ws[124]])
    clpw = jnp.stack([ws[119], ws[122], ws[125]]).astype(bf)

    hd_args = list(feats) + [base36, scal36,
        rldw, rlpw, rlpb, ws[126], ws[128].astype(bf), ws[127].reshape(1, 36),
        cldw, clpw, clpb, ws[114], ws[116].astype(bf), ws[115].reshape(1, 72)]
    hd_specs = [_batch_spec((h, w, FPN_CH)) for h, w in LVL_HW] + [
        _full_spec((5456, 36)), _full_spec((5456, 36)),
        _full_spec((3, 9, 64)), _full_spec((3, 64, 64)), _full_spec((3, 64)),
        _full_spec((9, 64)), _full_spec((64, 36)), _full_spec((1, 36)),
        _full_spec((3, 9, 64)), _full_spec((3, 64, 64)), _full_spec((3, 64)),
        _full_spec((9, 64)), _full_spec((64, 72)), _full_spec((1, 72))]
    bx, cl = pl.pallas_call(
        _heads_kernel,
        out_shape=(jax.ShapeDtypeStruct((B, 5456, 36), f32),
                   jax.ShapeDtypeStruct((B, 5456, 72), f32)),
        grid=(B,),
        in_specs=hd_specs,
        out_specs=(_batch_spec((5456, 36)), _batch_spec((5456, 72))),
        compiler_params=_cp(40 * 1024 * 1024),
    )(*hd_args)
    return bx, cl


@jax.jit
def _forward(ws, x, anchors):
    ya = (anchors[:, 0] + anchors[:, 2]) * 0.5
    xa = (anchors[:, 1] + anchors[:, 3]) * 0.5
    ha = anchors[:, 2] - anchors[:, 0]
    wa = anchors[:, 3] - anchors[:, 1]
    base36 = jnp.stack([xa, ya, xa, ya], axis=-1).reshape(5456, 36)
    scal36 = jnp.stack([wa, ha, wa, ha], axis=-1).reshape(5456, 36)
    w192 = _conv1_w192(ws[1]).astype(jnp.bfloat16)
    b128 = jnp.tile(ws[0], 4).reshape(1, 128)

    H = BATCH // 2
    bx0, cl0 = _half_pipeline(ws, x[:H], base36, scal36, w192, b128)
    bx1, cl1 = _half_pipeline(ws, x[H:], base36, scal36, w192, b128)
    bx = jnp.concatenate([bx0, bx1], axis=0)
    cl = jnp.concatenate([cl0, cl1], axis=0)

    A = anchors.shape[0]
    boxes = bx.reshape(BATCH, A, 4)
    classification = cl.reshape(BATCH, A, NUM_CLASSES)
    scores = jnp.max(classification, axis=-1, keepdims=True)
    return boxes, scores, classification


def kernel(w000, w001, w002, w003, w004, w005, w006, w007, w008, w009, w010, w011, w012, w013, w014, w015, w016, w017, w018, w019, w020, w021, w022, w023, w024, w025, w026, w027, w028, w029, w030, w031, w032, w033, w034, w035, w036, w037, w038, w039, w040, w041, w042, w043, w044, w045, w046, w047, w048, w049, w050, w051, w052, w053, w054, w055, w056, w057, w058, w059, w060, w061, w062, w063, w064, w065, w066, w067, w068, w069, w070, w071, w072, w073, w074, w075, w076, w077, w078, w079, w080, w081, w082, w083, w084, w085, w086, w087, w088, w089, w090, w091, w092, w093, w094, w095, w096, w097, w098, w099, w100, w101, w102, w103, w104, w105, w106, w107, w108, w109, w110, w111, w112, w113, w114, w115, w116, w117, w118, w119, w120, w121, w122, w123, w124, w125, w126, w127, w128, w129, w130, w131, w132, w133, w134, w135, w136, w137, x, anchors):
    ws = (w000, w001, w002, w003, w004, w005, w006, w007, w008, w009, w010, w011, w012, w013, w014, w015, w016, w017, w018, w019, w020, w021, w022, w023, w024, w025, w026, w027, w028, w029, w030, w031, w032, w033, w034, w035, w036, w037, w038, w039, w040, w041, w042, w043, w044, w045, w046, w047, w048, w049, w050, w051, w052, w053, w054, w055, w056, w057, w058, w059, w060, w061, w062, w063, w064, w065, w066, w067, w068, w069, w070, w071, w072, w073, w074, w075, w076, w077, w078, w079, w080, w081, w082, w083, w084, w085, w086, w087, w088, w089, w090, w091, w092, w093, w094, w095, w096, w097, w098, w099, w100, w101, w102, w103, w104, w105, w106, w107, w108, w109, w110, w111, w112, w113, w114, w115, w116, w117, w118, w119, w120, w121, w122, w123, w124, w125, w126, w127, w128, w129, w130, w131, w132, w133, w134, w135, w136, w137)
    return _forward(ws, x, anchors)


# speedup vs baseline: 1.5015x; 1.5015x over previous
import functools

import jax
import jax.numpy as jnp
from jax.experimental import pallas as pl
from jax.experimental.pallas import tpu as pltpu

BATCH = 8
IMG = 512
FPN_CH = 64
NUM_CLASSES = 8
LVL_HW = ((64, 64), (32, 32), (16, 16), (8, 8), (4, 4))


def _cp(vmem, n=1):
    return pltpu.CompilerParams(
        dimension_semantics=("parallel",) * n,
        vmem_limit_bytes=vmem)


_RS = ((1, 0), (0, 1), (1, 1))


def _conv3x3_s2(x, w_ref, b_ref, C):
    H = x.shape[0]
    oh = H // 2
    ph = x.reshape(oh, 2, oh, 2, C)
    PP = [[jnp.pad(ph[:, a, :, bb, :], ((1, 0), (1, 0), (0, 0)))
           for bb in range(2)] for a in range(2)]
    taps = []
    for p in range(3):
        a, sr = _RS[p]
        for q in range(3):
            bb, sc = _RS[q]
            taps.append(PP[a][bb][sr:sr + oh, sc:sc + oh, :])
    a2 = jnp.concatenate(taps, axis=-1).reshape(oh * oh, 9 * C)
    acc = jnp.dot(a2, w_ref[...], preferred_element_type=jnp.float32)
    acc = acc + b_ref[...]
    acc = acc * jax.nn.sigmoid(acc)
    return acc.astype(jnp.bfloat16).reshape(oh, oh, -1)


def _sep_node(xs, fuse, dw, pw, pb, pre_swish, post_act):
    H, W, C = xs[0].shape
    if fuse is not None:
        wk = [jnp.maximum(f, 0.0) for f in fuse]
        inv = 1.0 / (sum(wk) + 1e-4)
        s = (wk[0] * inv) * xs[0]
        for k in range(1, len(xs)):
            s = s + (wk[k] * inv) * xs[k]
    else:
        s = xs[0]
    if pre_swish:
        s = s * jax.nn.sigmoid(s)
    sp = jnp.pad(s, ((1, 1), (1, 1), (0, 0)))
    acc = None
    for k in range(9):
        dy, dx = divmod(k, 3)
        t = sp[dy:dy + H, dx:dx + W, :] * dw[k]
        acc = t if acc is None else acc + t
    out = jnp.dot(acc.reshape(H * W, C).astype(jnp.bfloat16), pw,
                  preferred_element_type=jnp.float32) + pb
    if post_act == "swish":
        out = out * jax.nn.sigmoid(out)
    elif post_act == "sigmoid":
        out = jax.nn.sigmoid(out)
    return out.reshape(H, W, -1)


def _pool(x):
    H, W, C = x.shape
    oh, ow = H // 2, W // 2
    ph = x.reshape(oh, 2, ow, 2, C)
    PP = [[jnp.pad(ph[:, a, :, bb, :],
                   ((0, 1), (0, 1), (0, 0)), constant_values=-jnp.inf)
           for bb in range(2)] for a in range(2)]
    RS = ((0, 0), (1, 0), (0, 1))
    o = None
    for p in range(3):
        a, sr = RS[p]
        for q in range(3):
            bb, sc = RS[q]
            t = PP[a][bb][sr:sr + oh, sc:sc + ow, :]
            o = t if o is None else jnp.maximum(o, t)
    return o


def _up(x):
    return jnp.repeat(jnp.repeat(x, 2, axis=0), 2, axis=1)


def _c1_kernel(xm_ref, xh_ref, w_ref, b_ref, o_ref):
    t = pl.program_id(1)
    halo = xh_ref[0]
    halo = jnp.where(t > 0, halo, jnp.zeros_like(halo))
    rows = jnp.concatenate([halo, xm_ref[0]], axis=0)
    rows = jnp.pad(rows, ((0, 0), (1, 0), (0, 0)))
    taps = [rows[di:di + 32, dj:dj + 128, :]
            for di in range(2) for dj in range(2)]
    a2 = jnp.concatenate(taps, axis=-1).reshape(32 * 128, 192)
    acc = jnp.dot(a2, w_ref[...], preferred_element_type=jnp.float32)
    acc = acc + b_ref[...]
    acc = acc * jax.nn.sigmoid(acc)
    o_ref[0] = acc.astype(jnp.bfloat16).reshape(32, 128, 128)


def _bb2_kernel(c_ref, w2_ref, b2_ref, w3_ref, b3_ref,
                w4_ref, b4_ref, w5_ref, b5_ref, p3_ref, p4_ref, p5_ref):
    cc = c_ref[0]
    pp = jnp.pad(cc, ((1, 0), (1, 0), (0, 0)))
    taps = []
    for p in range(3):
        a, sr = _RS[p]
        for q in range(3):
            bb, sc = _RS[q]
            g = (a * 2 + bb) * 32
            taps.append(pp[sr:sr + 128, sc:sc + 128, g:g + 32])
    a2 = jnp.concatenate(taps, axis=-1).reshape(128 * 128, 288)
    acc = jnp.dot(a2, w2_ref[...], preferred_element_type=jnp.float32)
    acc = acc + b2_ref[...]
    acc = acc * jax.nn.sigmoid(acc)
    c2 = acc.astype(jnp.bfloat16).reshape(128, 128, 24)
    p3 = _conv3x3_s2(c2, w3_ref, b3_ref, 24)
    p4 = _conv3x3_s2(p3, w4_ref, b4_ref, 40)
    p5 = _conv3x3_s2(p4, w5_ref, b5_ref, 112)
    p3_ref[0] = p3
    p4_ref[0] = p4
    p5_ref[0] = p5


def _bifpn_kernel(p3_ref, p4_ref, p5_ref, fw_ref,
                  p3dw_ref, p3db_ref, p4dw_ref, p4db_ref,
                  p5dw_ref, p5db_ref, p56w_ref, p56b_ref,
                  dw_ref, pw_ref, pb_ref,
                  o3_ref, o4_ref, o5_ref, o6_ref, o7_ref):
    def dense(x, w_ref2, b_ref2):
        H, W, C = x.shape
        o = jnp.dot(x.reshape(H * W, C), w_ref2[...],
                    preferred_element_type=jnp.float32) + b_ref2[...]
        return o.reshape(H, W, -1)

    p3_in = dense(p3_ref[0], p3dw_ref, p3db_ref)
    p4b = dense(p4_ref[0], p4dw_ref, p4db_ref)
    p5b = dense(p5_ref[0], p5dw_ref, p5db_ref)
    p5p6 = dense(p5_ref[0], p56w_ref, p56b_ref)
    p6_in = _pool(p5p6)
    p7_in = _pool(p6_in)

    f3 = f4 = f5 = f6 = f7 = None
    for c in range(3):
        base = 8 * c

        def F(off, n, c=c):
            return [fw_ref[19 * c + off + j] for j in range(n)]

        def node(xs, fuse, k, base=base):
            return _sep_node(xs, fuse, dw_ref[base + k], pw_ref[base + k],
                             pb_ref[base + k], True, "none")

        if c == 0:
            p4_1, p4_2 = p4b[..., :FPN_CH], p4b[..., FPN_CH:]
            p5_1, p5_2 = p5b[..., :FPN_CH], p5b[..., FPN_CH:]
            p3a, p6a, p7a = p3_in, p6_in, p7_in
        else:
            p3a, p4_1, p5_1, p6a, p7a = f3, f4, f5, f6, f7
            p4_2, p5_2 = p4_1, p5_1
        p6_td = node([p6a, _up(p7a)], F(0, 2), 0)
        p5_td = node([p5_1, _up(p6_td)], F(2, 2), 1)
        p4_td = node([p4_1, _up(p5_td)], F(4, 2), 2)
        f3 = node([p3a, _up(p4_td)], F(6, 2), 3)
        f4 = node([p4_2, p4_td, _pool(f3)], F(8, 3), 4)
        f5 = node([p5_2, p5_td, _pool(f4)], F(11, 3), 5)
        f6 = node([p6a, p6_td, _pool(f5)], F(14, 3), 6)
        f7 = node([p7a, _pool(f6)], F(17, 2), 7)

    o3_ref[0] = f3
    o4_ref[0] = f4
    o5_ref[0] = f5
    o6_ref[0] = f6
    o7_ref[0] = f7


_LVL_OFF = (0, 4096, 5120, 5376, 5440)


def _roll(x, k):
    return jnp.concatenate([x[:, -k:], x[:, :-k]], axis=1)


def _heads_kernel(f3_ref, f4_ref, f5_ref, f6_ref, f7_ref, base_ref, scal_ref,
                  rldw_ref, rlpw_ref, rlpb_ref, rhdw_ref, rhpw_ref, rhpb_ref,
                  cldw_ref, clpw_ref, clpb_ref, chdw_ref, chpw_ref, chpb_ref,
                  bx_ref, cl_ref):
    frefs = (f3_ref, f4_ref, f5_ref, f6_ref, f7_ref)
    for lvl in range(5):
        f = frefs[lvl][0]
        H, W, _ = f.shape
        r = f
        for l in range(3):
            r = _sep_node([r], None, rldw_ref[l], rlpw_ref[l], rlpb_ref[l],
                          False, "swish")
        r = _sep_node([r], None, rhdw_ref[...], rhpw_ref[...], rhpb_ref[...],
                      False, "none")
        R = r.reshape(H * W, 36)
        off = _LVL_OFF[lvl]
        sb = base_ref[off:off + H * W, :]
        sc = scal_ref[off:off + H * W, :]
        rm1, rp1 = _roll(R, -1), _roll(R, 1)
        rm3, rp3 = _roll(R, -3), _roll(R, 3)
        lane = jax.lax.broadcasted_iota(jnp.int32, (H * W, 36), 1) % 4
        dpos = jnp.where(lane == 0, rm1,
                         jnp.where(lane == 3, rp3, rp1))
        dsz = jnp.where(lane == 0, rm3,
                        jnp.where(lane == 3, rp1, rm1))
        center = dpos * sc + sb
        half = jnp.exp(dsz) * sc * 0.5
        lo = jnp.maximum(center - half, 0.0)
        hi = jnp.minimum(center + half, IMG - 1.0)
        bx_ref[0, off:off + H * W, :] = jnp.where(lane < 2, lo, hi)
        c = f
        for l in range(3):
            c = _sep_node([c], None, cldw_ref[l], clpw_ref[l], clpb_ref[l],
                          False, "swish")
        c = _sep_node([c], None, chdw_ref[...], chpw_ref[...], chpb_ref[...],
                      False, "sigmoid")
        cl_ref[0, off:off + H * W, :] = c.reshape(H * W, 72)


def _conv1_w192(w1):
    rows, grps, srcs = [], [], []
    for al in range(2):
        for p in range(3):
            di, a4 = (0, 3) if (al == 0 and p == 0) else (1, 2 * al + p - 1)
            for be in range(2):
                for q in range(3):
                    dj, b4 = ((0, 3) if (be == 0 and q == 0)
                              else (1, 2 * be + q - 1))
                    for cch in range(3):
                        rows.append((di * 2 + dj) * 48 + a4 * 12 + b4 * 3
                                    + cch)
                        grps.append(al * 2 + be)
                        srcs.append((p * 3 + q) * 3 + cch)
    w4 = jnp.zeros((192, 4, 32), jnp.float32)
    w4 = w4.at[jnp.asarray(rows), jnp.asarray(grps)].set(w1[jnp.asarray(srcs)])
    return w4.reshape(192, 128)


def _full_spec(shape):
    nd = len(shape)
    return pl.BlockSpec(shape, lambda b, nd=nd: (0,) * nd)


def _batch_spec(shape):
    nd = len(shape)
    return pl.BlockSpec((1,) + tuple(shape), lambda b, nd=nd: (b,) + (0,) * nd)


@jax.jit
def _forward(ws, x, anchors):
    bf = jnp.bfloat16
    f32 = jnp.float32
    B = BATCH

    xs = jnp.transpose(x.astype(bf).reshape(B, 3, 128, 4, 128, 4),
                       (0, 2, 4, 3, 5, 1)).reshape(B, 128, 128, 48)
    b128 = jnp.tile(ws[0], 4).reshape(1, 128)
    c1 = pl.pallas_call(
        _c1_kernel,
        out_shape=jax.ShapeDtypeStruct((B, 128, 128, 128), bf),
        grid=(B, 4),
        in_specs=[
            pl.BlockSpec((1, 32, 128, 48), lambda b, t: (b, t, 0, 0)),
            pl.BlockSpec((1, 1, 128, 48),
                         lambda b, t: (b, jnp.maximum(32 * t - 1, 0), 0, 0)),
            pl.BlockSpec((192, 128), lambda b, t: (0, 0)),
            pl.BlockSpec((1, 128), lambda b, t: (0, 0))],
        out_specs=pl.BlockSpec((1, 32, 128, 128), lambda b, t: (b, t, 0, 0)),
        compiler_params=_cp(40 * 1024 * 1024, 2),
    )(xs, xs, _conv1_w192(ws[1]).astype(bf), b128)

    bb_args = [c1]
    bb_specs = [_batch_spec((128, 128, 128))]
    for wi, bi in ((3, 2), (5, 4), (7, 6), (9, 8)):
        bb_args += [ws[wi].astype(bf), ws[bi].reshape(1, -1)]
        bb_specs += [_full_spec(ws[wi].shape), _full_spec((1, ws[bi].shape[0]))]
    p3, p4, p5 = pl.pallas_call(
        _bb2_kernel,
        out_shape=(jax.ShapeDtypeStruct((B, 64, 64, 40), bf),
                   jax.ShapeDtypeStruct((B, 32, 32, 112), bf),
                   jax.ShapeDtypeStruct((B, 16, 16, 320), bf)),
        grid=(B,),
        in_specs=bb_specs,
        out_specs=(_batch_spec((64, 64, 40)), _batch_spec((32, 32, 112)),
                   _batch_spec((16, 16, 320))),
        compiler_params=_cp(60 * 1024 * 1024),
    )(*bb_args)

    sep_off = {"conv3_up": 0, "conv4_down": 3, "conv4_up": 6, "conv5_down": 9,
               "conv5_up": 12, "conv6_down": 15, "conv6_up": 18,
               "conv7_down": 21}
    node_order = ("conv6_up", "conv5_up", "conv4_up", "conv3_up",
                  "conv4_down", "conv5_down", "conv6_down", "conv7_down")
    fuse_off = {"w3_up": 24, "w4_dn": 25, "w4_up": 26, "w5_dn": 27,
                "w5_up": 28, "w6_dn": 29, "w6_up": 30, "w7_dn": 31}
    fuse_order = ("w6_up", "w5_up", "w4_up", "w3_up", "w4_dn", "w5_dn",
                  "w6_dn", "w7_dn")
    DW, PW, PB, FW = [], [], [], []
    for c in range(3):
        base = 10 + 32 * c
        for name in node_order:
            o = base + sep_off[name]
            DW.append(ws[o])
            PB.append(ws[o + 1])
            PW.append(ws[o + 2])
        for name in fuse_order:
            FW.append(ws[base + fuse_off[name]])
    DW = jnp.stack(DW)
    PW = jnp.stack(PW).astype(bf)
    PB = jnp.stack(PB)
    FW = jnp.concatenate(FW)

    fp_args = [p3, p4, p5, FW,
               ws[107].astype(bf), ws[106].reshape(1, 64),
               ws[109].astype(bf), ws[108].reshape(1, 128),
               ws[111].astype(bf), ws[110].reshape(1, 128),
               ws[113].astype(bf), ws[112].reshape(1, 64),
               DW, PW, PB]
    fp_specs = [_batch_spec((64, 64, 40)), _batch_spec((32, 32, 112)),
                _batch_spec((16, 16, 320)),
                pl.BlockSpec(memory_space=pltpu.MemorySpace.SMEM),
                _full_spec((40, 64)), _full_spec((1, 64)),
                _full_spec((112, 128)), _full_spec((1, 128)),
                _full_spec((320, 128)), _full_spec((1, 128)),
                _full_spec((320, 64)), _full_spec((1, 64)),
                _full_spec((24, 9, 64)), _full_spec((24, 64, 64)),
                _full_spec((24, 64))]
    feats = pl.pallas_call(
        _bifpn_kernel,
        out_shape=tuple(jax.ShapeDtypeStruct((B, h, w, FPN_CH), f32)
                        for h, w in LVL_HW),
        grid=(B,),
        in_specs=fp_specs,
        out_specs=tuple(_batch_spec((h, w, FPN_CH)) for h, w in LVL_HW),
        compiler_params=_cp(40 * 1024 * 1024),
    )(*fp_args)

    rldw = jnp.stack([ws[129], ws[132], ws[135]])
    rlpb = jnp.stack([ws[130], ws[133], ws[136]])
    rlpw = jnp.stack([ws[131], ws[134], ws[137]]).astype(bf)
    cldw = jnp.stack([ws[117], ws[120], ws[123]])
    clpb = jnp.stack([ws[118], ws[121], ws[124]])
    clpw = jnp.stack([ws[119], ws[122], ws[125]]).astype(bf)
    ya = (anchors[:, 0] + anchors[:, 2]) * 0.5
    xa = (anchors[:, 1] + anchors[:, 3]) * 0.5
    ha = anchors[:, 2] - anchors[:, 0]
    wa = anchors[:, 3] - anchors[:, 1]
    base36 = jnp.stack([xa, ya, xa, ya], axis=-1).reshape(5456, 36)
    scal36 = jnp.stack([wa, ha, wa, ha], axis=-1).reshape(5456, 36)

    hd_args = list(feats) + [base36, scal36,
        rldw, rlpw, rlpb, ws[126], ws[128].astype(bf), ws[127].reshape(1, 36),
        cldw, clpw, clpb, ws[114], ws[116].astype(bf), ws[115].reshape(1, 72)]
    hd_specs = [_batch_spec((h, w, FPN_CH)) for h, w in LVL_HW] + [
        _full_spec((5456, 36)), _full_spec((5456, 36)),
        _full_spec((3, 9, 64)), _full_spec((3, 64, 64)), _full_spec((3, 64)),
        _full_spec((9, 64)), _full_spec((64, 36)), _full_spec((1, 36)),
        _full_spec((3, 9, 64)), _full_spec((3, 64, 64)), _full_spec((3, 64)),
        _full_spec((9, 64)), _full_spec((64, 72)), _full_spec((1, 72))]
    bx, cl = pl.pallas_call(
        _heads_kernel,
        out_shape=(jax.ShapeDtypeStruct((B, 5456, 36), f32),
                   jax.ShapeDtypeStruct((B, 5456, 72), f32)),
        grid=(B,),
        in_specs=hd_specs,
        out_specs=(_batch_spec((5456, 36)), _batch_spec((5456, 72))),
        compiler_params=_cp(40 * 1024 * 1024),
    )(*hd_args)

    A = anchors.shape[0]
    boxes = bx.reshape(B, A, 4)
    classification = cl.reshape(B, A, NUM_CLASSES)
    scores = jnp.max(classification, axis=-1, keepdims=True)
    return boxes, scores, classification


def kernel(w000, w001, w002, w003, w004, w005, w006, w007, w008, w009, w010, w011, w012, w013, w014, w015, w016, w017, w018, w019, w020, w021, w022, w023, w024, w025, w026, w027, w028, w029, w030, w031, w032, w033, w034, w035, w036, w037, w038, w039, w040, w041, w042, w043, w044, w045, w046, w047, w048, w049, w050, w051, w052, w053, w054, w055, w056, w057, w058, w059, w060, w061, w062, w063, w064, w065, w066, w067, w068, w069, w070, w071, w072, w073, w074, w075, w076, w077, w078, w079, w080, w081, w082, w083, w084, w085, w086, w087, w088, w089, w090, w091, w092, w093, w094, w095, w096, w097, w098, w099, w100, w101, w102, w103, w104, w105, w106, w107, w108, w109, w110, w111, w112, w113, w114, w115, w116, w117, w118, w119, w120, w121, w122, w123, w124, w125, w126, w127, w128, w129, w130, w131, w132, w133, w134, w135, w136, w137, x, anchors):
    ws = (w000, w001, w002, w003, w004, w005, w006, w007, w008, w009, w010, w011, w012, w013, w014, w015, w016, w017, w018, w019, w020, w021, w022, w023, w024, w025, w026, w027, w028, w029, w030, w031, w032, w033, w034, w035, w036, w037, w038, w039, w040, w041, w042, w043, w044, w045, w046, w047, w048, w049, w050, w051, w052, w053, w054, w055, w056, w057, w058, w059, w060, w061, w062, w063, w064, w065, w066, w067, w068, w069, w070, w071, w072, w073, w074, w075, w076, w077, w078, w079, w080, w081, w082, w083, w084, w085, w086, w087, w088, w089, w090, w091, w092, w093, w094, w095, w096, w097, w098, w099, w100, w101, w102, w103, w104, w105, w106, w107, w108, w109, w110, w111, w112, w113, w114, w115, w116, w117, w118, w119, w120, w121, w122, w123, w124, w125, w126, w127, w128, w129, w130, w131, w132, w133, w134, w135, w136, w137)
    return _forward(ws, x, anchors)
